# Initial kernel scaffold; baseline (speedup 1.0000x reference)
import jax
import jax.numpy as jnp
from jax.experimental import pallas as pl


def kernel(xyz_in, pts_in, d0w, d0b, d1w, d1b, d2w, d2b, m0w, m0b, m1w, m1b, v0w, v0b, v1w, v1b, v2w, v2b, lw, lb):
    raise NotImplementedError("write your pallas kernel here")


# speedup vs baseline: 1.0017x; 1.0017x over previous
import functools

import jax
import jax.numpy as jnp
from jax import lax
from jax.experimental import pallas as pl
from jax.experimental.pallas import tpu as pltpu

_VMEM_LIMIT = 48 * 1024 * 1024
_LOG2E = 1.4426950408889634


def _density_kernel(xyz_ref, tile_ref, w0_ref, b0_ref, w1_ref, b1_ref,
                    w2_ref, b2_ref, out_ref, *, nl2, post_scale):
    xa = xyz_ref[...]
    xr = tile_ref[...]
    xx = lax.dot_general(xr, xa, (((1,), (1,)), ((), ())),
                         preferred_element_type=jnp.float32)
    ra = jnp.sum(xr * xr, axis=-1, keepdims=True) * nl2
    aa = (jnp.sum(xa * xa, axis=-1) * nl2)[None, :]
    arg = jnp.minimum(ra + aa + (-2.0 * nl2) * xx, 0.0)
    g = jnp.exp2(arg)
    d = jnp.sum(g, axis=-1, keepdims=True) * post_scale

    h = jnp.maximum(d * w0_ref[...] + b0_ref[...], 0.0)
    h = jnp.maximum(jnp.dot(h, w1_ref[...],
                            preferred_element_type=jnp.float32) + b1_ref[...], 0.0)
    h = jnp.maximum(jnp.dot(h, w2_ref[...],
                            preferred_element_type=jnp.float32) + b2_ref[...], 0.0)
    out_ref[...] = h


def _density_scale(xyz, dnet, bandwidth, *, tn=512):
    B, N, _ = xyz.shape
    (w0, b0), (w1, b1), (w2, b2) = dnet
    neg = -1.0 / (2.0 * bandwidth * bandwidth)
    kfn = functools.partial(
        _density_kernel, nl2=neg * _LOG2E,
        post_scale=1.0 / (float(N) * 2.5 * bandwidth))

    def wspec(a):
        nd = a.ndim
        return pl.BlockSpec(a.shape, lambda b, t, nd=nd: (0,) * nd)

    return pl.pallas_call(
        kfn,
        out_shape=jax.ShapeDtypeStruct((B, N, 1), jnp.float32),
        grid=(B, N // tn),
        in_specs=[
            pl.BlockSpec((None, N, 3), lambda b, t: (b, 0, 0)),
            pl.BlockSpec((None, tn, 3), lambda b, t: (b, t, 0)),
            wspec(w0), wspec(b0), wspec(w1), wspec(b1), wspec(w2), wspec(b2),
        ],
        out_specs=pl.BlockSpec((None, tn, 1), lambda b, t: (b, t, 0)),
        compiler_params=pltpu.CompilerParams(
            dimension_semantics=("parallel", "parallel"),
            vmem_limit_bytes=_VMEM_LIMIT),
    )(xyz, xyz, w0, b0, w1, b1, w2, b2)


def _farthest_point_sample(xyz, npoint):
    B, N, _ = xyz.shape

    def body(i, state):
        distance, farthest, centroids = state
        centroids = centroids.at[:, i].set(farthest)
        centroid = _gather_points(xyz, farthest[:, None])
        dist = jnp.sum((xyz - centroid) ** 2, -1)
        distance = jnp.minimum(distance, dist)
        farthest = jnp.argmax(distance, axis=-1).astype(jnp.int32)
        return distance, farthest, centroids

    state = (jnp.full((B, N), 1e10, jnp.float32),
             jnp.zeros((B,), jnp.int32),
             jnp.zeros((B, npoint), jnp.int32))
    _, _, centroids = lax.fori_loop(0, npoint, body, state)
    return centroids


def _pointconv_kernel(x_ref, mw0_ref, mb0_ref, mw1_ref, mb1_ref,
                      vw0_ref, vb0_ref, vw1_ref, vb1_ref, vw2_ref, vb2_ref,
                      wl_ref, bl_ref, out_ref, *, k):
    skn, c_all = x_ref.shape
    s_t = skn // k
    cm = mw1_ref.shape[1]
    wc = vw2_ref.shape[1]

    x = x_ref[...]

    f = jnp.maximum(jnp.dot(x, mw0_ref[...],
                            preferred_element_type=jnp.float32) + mb0_ref[...], 0.0)
    f = jnp.maximum(jnp.dot(f, mw1_ref[...],
                            preferred_element_type=jnp.float32) + mb1_ref[...], 0.0)

    w = jnp.maximum(jnp.dot(x, vw0_ref[...],
                            preferred_element_type=jnp.float32) + vb0_ref[...], 0.0)
    w = jnp.maximum(jnp.dot(w, vw1_ref[...],
                            preferred_element_type=jnp.float32) + vb1_ref[...], 0.0)
    w = jnp.maximum(jnp.dot(w, vw2_ref[...],
                            preferred_element_type=jnp.float32) + vb2_ref[...], 0.0)

    wd = w * x[:, c_all - 1:c_all]

    f3t = jnp.swapaxes(f.reshape(s_t, k, cm), 1, 2)
    wd3 = wd.reshape(s_t, k, wc)
    agg = jnp.einsum("sck,skw->scw", f3t, wd3,
                     preferred_element_type=jnp.float32)

    flat = agg.reshape(s_t, cm * wc)
    out = jnp.dot(flat, wl_ref[...], preferred_element_type=jnp.float32) + bl_ref[...]
    out_ref[...] = jnp.maximum(out, 0.0)


def _pointconv(grouped_all, mlp, wnet, linear, *, s_tile=256):
    B, S, K, C_all = grouped_all.shape
    (mw0, mb0), (mw1, mb1) = mlp
    (vw0, vb0), (vw1, vb1), (vw2, vb2) = wnet
    wl, bl = linear
    CO = wl.shape[1]

    mw0_ext = jnp.concatenate(
        [mw0, jnp.zeros((C_all - mw0.shape[0], mw0.shape[1]), mw0.dtype)], axis=0)
    vw0_ext = jnp.concatenate(
        [vw0, jnp.zeros((C_all - vw0.shape[0], vw0.shape[1]), vw0.dtype)], axis=0)

    x_flat = grouped_all.reshape(B, S * K, C_all)
    kfn = functools.partial(_pointconv_kernel, k=K)

    def wspec(a):
        nd = a.ndim
        return pl.BlockSpec(a.shape, lambda b, st, nd=nd: (0,) * nd)

    return pl.pallas_call(
        kfn,
        out_shape=jax.ShapeDtypeStruct((B, S, CO), jnp.float32),
        grid=(B, S // s_tile),
        in_specs=[
            pl.BlockSpec((None, s_tile * K, C_all), lambda b, st: (b, st, 0)),
            wspec(mw0_ext), wspec(mb0), wspec(mw1), wspec(mb1),
            wspec(vw0_ext), wspec(vb0), wspec(vw1), wspec(vb1),
            wspec(vw2), wspec(vb2), wspec(wl), wspec(bl),
        ],
        out_specs=pl.BlockSpec((None, s_tile, CO), lambda b, st: (b, st, 0)),
        compiler_params=pltpu.CompilerParams(
            dimension_semantics=("parallel", "parallel"),
            vmem_limit_bytes=_VMEM_LIMIT),
    )(x_flat, mw0_ext, mb0, mw1, mb1, vw0_ext, vb0, vw1, vb1, vw2, vb2, wl, bl)


def _gather_points(points, idx):
    return jax.vmap(lambda p, i: p[i])(points, idx)


def kernel(xyz_in, pts_in,
           d0w, d0b, d1w, d1b, d2w, d2b,
           m0w, m0b, m1w, m1b,
           v0w, v0b, v1w, v1b, v2w, v2b,
           lw, lb):
    npoint, nsample, bandwidth = 512, 32, 0.5
    xyz = jnp.transpose(xyz_in, (0, 2, 1))
    points = jnp.transpose(pts_in, (0, 2, 1))

    density_scale = _density_scale(
        xyz, [(d0w, d0b), (d1w, d1b), (d2w, d2b)], bandwidth)

    fps_idx = _farthest_point_sample(xyz, npoint)
    new_xyz = _gather_points(xyz, fps_idx)

    sqr = -2.0 * jnp.einsum("bnc,bmc->bnm", new_xyz, xyz)
    sqr = sqr + jnp.sum(new_xyz ** 2, -1)[:, :, None]
    sqr = sqr + jnp.sum(xyz ** 2, -1)[:, None, :]
    _, idx = lax.top_k(-sqr, nsample)

    grouped_xyz = _gather_points(xyz, idx)
    grouped_xyz_norm = grouped_xyz - new_xyz[:, :, None, :]
    grouped_points = _gather_points(points, idx)
    grouped_density = _gather_points(density_scale, idx)
    grouped_all = jnp.concatenate(
        [grouped_xyz_norm, grouped_points, grouped_density], axis=-1)

    feat = _pointconv(grouped_all,
                      [(m0w, m0b), (m1w, m1b)],
                      [(v0w, v0b), (v1w, v1b), (v2w, v2b)],
                      (lw, lb))
    return jnp.transpose(new_xyz, (0, 2, 1)), jnp.transpose(feat, (0, 2, 1))


# speedup vs baseline: 1.2608x; 1.2587x over previous
import functools

import jax
import jax.numpy as jnp
from jax import lax
from jax.experimental import pallas as pl
from jax.experimental.pallas import tpu as pltpu

_VMEM_LIMIT = 48 * 1024 * 1024
_LOG2E = 1.4426950408889634


def _density_kernel(xyz_ref, tile_ref, w0_ref, b0_ref, w1_ref, b1_ref,
                    w2_ref, b2_ref, out_ref, *, nl2, post_scale):
    xa = xyz_ref[...]
    xr = tile_ref[...]
    xx = lax.dot_general(xr, xa, (((1,), (1,)), ((), ())),
                         preferred_element_type=jnp.float32)
    ra = jnp.sum(xr * xr, axis=-1, keepdims=True) * nl2
    aa = (jnp.sum(xa * xa, axis=-1) * nl2)[None, :]
    arg = jnp.minimum(ra + aa + (-2.0 * nl2) * xx, 0.0)
    g = jnp.exp2(arg)
    d = jnp.sum(g, axis=-1, keepdims=True) * post_scale

    h = jnp.maximum(d * w0_ref[...] + b0_ref[...], 0.0)
    h = jnp.maximum(jnp.dot(h, w1_ref[...],
                            preferred_element_type=jnp.float32) + b1_ref[...], 0.0)
    h = jnp.maximum(jnp.dot(h, w2_ref[...],
                            preferred_element_type=jnp.float32) + b2_ref[...], 0.0)
    out_ref[...] = h


def _density_scale(xyz, dnet, bandwidth, *, tn=512):
    B, N, _ = xyz.shape
    (w0, b0), (w1, b1), (w2, b2) = dnet
    neg = -1.0 / (2.0 * bandwidth * bandwidth)
    kfn = functools.partial(
        _density_kernel, nl2=neg * _LOG2E,
        post_scale=1.0 / (float(N) * 2.5 * bandwidth))

    def wspec(a):
        nd = a.ndim
        return pl.BlockSpec(a.shape, lambda b, t, nd=nd: (0,) * nd)

    return pl.pallas_call(
        kfn,
        out_shape=jax.ShapeDtypeStruct((B, N, 1), jnp.float32),
        grid=(B, N // tn),
        in_specs=[
            pl.BlockSpec((None, N, 3), lambda b, t: (b, 0, 0)),
            pl.BlockSpec((None, tn, 3), lambda b, t: (b, t, 0)),
            wspec(w0), wspec(b0), wspec(w1), wspec(b1), wspec(w2), wspec(b2),
        ],
        out_specs=pl.BlockSpec((None, tn, 1), lambda b, t: (b, t, 0)),
        compiler_params=pltpu.CompilerParams(
            dimension_semantics=("parallel", "parallel"),
            vmem_limit_bytes=_VMEM_LIMIT),
    )(xyz, xyz, w0, b0, w1, b1, w2, b2)


def _fps_kernel(xs_ref, ys_ref, zs_ref, out_ref, *, npoint, n):
    bt = xs_ref.shape[0]
    xs = xs_ref[...]
    ys = ys_ref[...]
    zs = zs_ref[...]
    iota = lax.broadcasted_iota(jnp.int32, (bt, n), 1)

    def body(i, carry):
        dmin, far = carry
        out_ref[pl.ds(i, 1), :, :] = far[None]
        mask = iota == far
        cx = jnp.sum(jnp.where(mask, xs, 0.0), axis=1, keepdims=True)
        cy = jnp.sum(jnp.where(mask, ys, 0.0), axis=1, keepdims=True)
        cz = jnp.sum(jnp.where(mask, zs, 0.0), axis=1, keepdims=True)
        dx = xs - cx
        dy = ys - cy
        dz = zs - cz
        dist = (dx * dx + dz * dz) + dy * dy
        dmin = jnp.minimum(dmin, dist)
        m = jnp.max(dmin, axis=1, keepdims=True)
        far = jnp.min(jnp.where(dmin == m, iota, n), axis=1, keepdims=True)
        return dmin, far

    dmin0 = jnp.full((bt, n), 1e10, jnp.float32)
    far0 = jnp.zeros((bt, 1), jnp.int32)
    lax.fori_loop(0, npoint, body, (dmin0, far0))


def _farthest_point_sample(xyz_b3n, npoint, *, bt=8):
    B, _, N = xyz_b3n.shape
    kfn = functools.partial(_fps_kernel, npoint=npoint, n=N)
    out = pl.pallas_call(
        kfn,
        out_shape=jax.ShapeDtypeStruct((B // bt, npoint, bt, 1), jnp.int32),
        grid=(B // bt,),
        in_specs=[pl.BlockSpec((bt, N), lambda g: (g, 0))] * 3,
        out_specs=pl.BlockSpec((None, npoint, bt, 1), lambda g: (g, 0, 0, 0)),
        compiler_params=pltpu.CompilerParams(
            dimension_semantics=("parallel",),
            vmem_limit_bytes=_VMEM_LIMIT),
    )(xyz_b3n[:, 0, :], xyz_b3n[:, 1, :], xyz_b3n[:, 2, :])
    return out[..., 0].transpose(0, 2, 1).reshape(B, npoint)


def _pointconv_kernel(x_ref, mw0_ref, mb0_ref, mw1_ref, mb1_ref,
                      vw0_ref, vb0_ref, vw1_ref, vb1_ref, vw2_ref, vb2_ref,
                      wl_ref, bl_ref, out_ref, *, k):
    skn, c_all = x_ref.shape
    s_t = skn // k
    cm = mw1_ref.shape[1]
    wc = vw2_ref.shape[1]

    x = x_ref[...]

    f = jnp.maximum(jnp.dot(x, mw0_ref[...],
                            preferred_element_type=jnp.float32) + mb0_ref[...], 0.0)
    f = jnp.maximum(jnp.dot(f, mw1_ref[...],
                            preferred_element_type=jnp.float32) + mb1_ref[...], 0.0)

    w = jnp.maximum(jnp.dot(x, vw0_ref[...],
                            preferred_element_type=jnp.float32) + vb0_ref[...], 0.0)
    w = jnp.maximum(jnp.dot(w, vw1_ref[...],
                            preferred_element_type=jnp.float32) + vb1_ref[...], 0.0)
    w = jnp.maximum(jnp.dot(w, vw2_ref[...],
                            preferred_element_type=jnp.float32) + vb2_ref[...], 0.0)

    wd = w * x[:, c_all - 1:c_all]

    f3t = jnp.swapaxes(f.reshape(s_t, k, cm), 1, 2)
    wd3 = wd.reshape(s_t, k, wc)
    agg = jnp.einsum("sck,skw->scw", f3t, wd3,
                     preferred_element_type=jnp.float32)

    flat = agg.reshape(s_t, cm * wc)
    out = jnp.dot(flat, wl_ref[...], preferred_element_type=jnp.float32) + bl_ref[...]
    out_ref[...] = jnp.maximum(out, 0.0)


def _pointconv(grouped_all, mlp, wnet, linear, *, s_tile=256):
    B, S, K, C_all = grouped_all.shape
    (mw0, mb0), (mw1, mb1) = mlp
    (vw0, vb0), (vw1, vb1), (vw2, vb2) = wnet
    wl, bl = linear
    CO = wl.shape[1]

    mw0_ext = jnp.concatenate(
        [mw0, jnp.zeros((C_all - mw0.shape[0], mw0.shape[1]), mw0.dtype)], axis=0)
    vw0_ext = jnp.concatenate(
        [vw0, jnp.zeros((C_all - vw0.shape[0], vw0.shape[1]), vw0.dtype)], axis=0)

    x_flat = grouped_all.reshape(B, S * K, C_all)
    kfn = functools.partial(_pointconv_kernel, k=K)

    def wspec(a):
        nd = a.ndim
        return pl.BlockSpec(a.shape, lambda b, st, nd=nd: (0,) * nd)

    return pl.pallas_call(
        kfn,
        out_shape=jax.ShapeDtypeStruct((B, S, CO), jnp.float32),
        grid=(B, S // s_tile),
        in_specs=[
            pl.BlockSpec((None, s_tile * K, C_all), lambda b, st: (b, st, 0)),
            wspec(mw0_ext), wspec(mb0), wspec(mw1), wspec(mb1),
            wspec(vw0_ext), wspec(vb0), wspec(vw1), wspec(vb1),
            wspec(vw2), wspec(vb2), wspec(wl), wspec(bl),
        ],
        out_specs=pl.BlockSpec((None, s_tile, CO), lambda b, st: (b, st, 0)),
        compiler_params=pltpu.CompilerParams(
            dimension_semantics=("parallel", "parallel"),
            vmem_limit_bytes=_VMEM_LIMIT),
    )(x_flat, mw0_ext, mb0, mw1, mb1, vw0_ext, vb0, vw1, vb1, vw2, vb2, wl, bl)


def _gather_points(points, idx):
    return jax.vmap(lambda p, i: p[i])(points, idx)


def kernel(xyz_in, pts_in,
           d0w, d0b, d1w, d1b, d2w, d2b,
           m0w, m0b, m1w, m1b,
           v0w, v0b, v1w, v1b, v2w, v2b,
           lw, lb):
    npoint, nsample, bandwidth = 512, 32, 0.5
    xyz = jnp.transpose(xyz_in, (0, 2, 1))
    points = jnp.transpose(pts_in, (0, 2, 1))

    density_scale = _density_scale(
        xyz, [(d0w, d0b), (d1w, d1b), (d2w, d2b)], bandwidth)

    fps_idx = _farthest_point_sample(xyz_in, npoint)
    new_xyz = _gather_points(xyz, fps_idx)

    sqr = -2.0 * jnp.einsum("bnc,bmc->bnm", new_xyz, xyz)
    sqr = sqr + jnp.sum(new_xyz ** 2, -1)[:, :, None]
    sqr = sqr + jnp.sum(xyz ** 2, -1)[:, None, :]
    _, idx = lax.top_k(-sqr, nsample)

    grouped_xyz = _gather_points(xyz, idx)
    grouped_xyz_norm = grouped_xyz - new_xyz[:, :, None, :]
    grouped_points = _gather_points(points, idx)
    grouped_density = _gather_points(density_scale, idx)
    grouped_all = jnp.concatenate(
        [grouped_xyz_norm, grouped_points, grouped_density], axis=-1)

    feat = _pointconv(grouped_all,
                      [(m0w, m0b), (m1w, m1b)],
                      [(v0w, v0b), (v1w, v1b), (v2w, v2b)],
                      (lw, lb))
    return jnp.transpose(new_xyz, (0, 2, 1)), jnp.transpose(feat, (0, 2, 1))


# speedup vs baseline: 2.0523x; 1.6277x over previous
import functools

import jax
import jax.numpy as jnp
from jax import lax
from jax.experimental import pallas as pl
from jax.experimental.pallas import tpu as pltpu

_VMEM_LIMIT = 48 * 1024 * 1024
_LOG2E = 1.4426950408889634


def _density_kernel(xyz_ref, tile_ref, w0_ref, b0_ref, w1_ref, b1_ref,
                    w2_ref, b2_ref, out_ref, *, nl2, post_scale):
    xa = xyz_ref[...]
    xr = tile_ref[...]
    xx = lax.dot_general(xr, xa, (((1,), (1,)), ((), ())),
                         preferred_element_type=jnp.float32)
    ra = jnp.sum(xr * xr, axis=-1, keepdims=True) * nl2
    aa = (jnp.sum(xa * xa, axis=-1) * nl2)[None, :]
    arg = jnp.minimum(ra + aa + (-2.0 * nl2) * xx, 0.0)
    g = jnp.exp2(arg)
    d = jnp.sum(g, axis=-1, keepdims=True) * post_scale

    h = jnp.maximum(d * w0_ref[...] + b0_ref[...], 0.0)
    h = jnp.maximum(jnp.dot(h, w1_ref[...],
                            preferred_element_type=jnp.float32) + b1_ref[...], 0.0)
    h = jnp.maximum(jnp.dot(h, w2_ref[...],
                            preferred_element_type=jnp.float32) + b2_ref[...], 0.0)
    out_ref[...] = h


def _density_scale(xyz, dnet, bandwidth, *, tn=512):
    B, N, _ = xyz.shape
    (w0, b0), (w1, b1), (w2, b2) = dnet
    neg = -1.0 / (2.0 * bandwidth * bandwidth)
    kfn = functools.partial(
        _density_kernel, nl2=neg * _LOG2E,
        post_scale=1.0 / (float(N) * 2.5 * bandwidth))

    def wspec(a):
        nd = a.ndim
        return pl.BlockSpec(a.shape, lambda b, t, nd=nd: (0,) * nd)

    return pl.pallas_call(
        kfn,
        out_shape=jax.ShapeDtypeStruct((B, N, 1), jnp.float32),
        grid=(B, N // tn),
        in_specs=[
            pl.BlockSpec((None, N, 3), lambda b, t: (b, 0, 0)),
            pl.BlockSpec((None, tn, 3), lambda b, t: (b, t, 0)),
            wspec(w0), wspec(b0), wspec(w1), wspec(b1), wspec(w2), wspec(b2),
        ],
        out_specs=pl.BlockSpec((None, tn, 1), lambda b, t: (b, t, 0)),
        compiler_params=pltpu.CompilerParams(
            dimension_semantics=("parallel", "parallel"),
            vmem_limit_bytes=_VMEM_LIMIT),
    )(xyz, xyz, w0, b0, w1, b1, w2, b2)


def _fps_kernel(xs_ref, ys_ref, zs_ref, out_ref, *, npoint, n):
    bt = xs_ref.shape[0]
    xs = xs_ref[...]
    ys = ys_ref[...]
    zs = zs_ref[...]
    iota = lax.broadcasted_iota(jnp.int32, (bt, n), 1)

    def body(i, carry):
        dmin, far = carry
        out_ref[pl.ds(i, 1), :, :] = far[None]
        mask = iota == far
        cx = jnp.sum(jnp.where(mask, xs, 0.0), axis=1, keepdims=True)
        cy = jnp.sum(jnp.where(mask, ys, 0.0), axis=1, keepdims=True)
        cz = jnp.sum(jnp.where(mask, zs, 0.0), axis=1, keepdims=True)
        dx = xs - cx
        dy = ys - cy
        dz = zs - cz
        dist = (dx * dx + dz * dz) + dy * dy
        dmin = jnp.minimum(dmin, dist)
        m = jnp.max(dmin, axis=1, keepdims=True)
        far = jnp.min(jnp.where(dmin == m, iota, n), axis=1, keepdims=True)
        return dmin, far

    dmin0 = jnp.full((bt, n), 1e10, jnp.float32)
    far0 = jnp.zeros((bt, 1), jnp.int32)
    lax.fori_loop(0, npoint, body, (dmin0, far0))


def _farthest_point_sample(xyz_b3n, npoint, *, bt=8):
    B, _, N = xyz_b3n.shape
    kfn = functools.partial(_fps_kernel, npoint=npoint, n=N)
    out = pl.pallas_call(
        kfn,
        out_shape=jax.ShapeDtypeStruct((B // bt, npoint, bt, 1), jnp.int32),
        grid=(B // bt,),
        in_specs=[pl.BlockSpec((bt, N), lambda g: (g, 0))] * 3,
        out_specs=pl.BlockSpec((None, npoint, bt, 1), lambda g: (g, 0, 0, 0)),
        compiler_params=pltpu.CompilerParams(
            dimension_semantics=("parallel",),
            vmem_limit_bytes=_VMEM_LIMIT),
    )(xyz_b3n[:, 0, :], xyz_b3n[:, 1, :], xyz_b3n[:, 2, :])
    return out[..., 0].transpose(0, 2, 1).reshape(B, npoint)


def _knn_kernel(sqr_ref, out_ref, *, nsample, n):
    d = sqr_ref[...]
    st = d.shape[0]
    iota = lax.broadcasted_iota(jnp.int32, (st, n), 1)
    for k in range(nsample):
        m = jnp.min(d, axis=1, keepdims=True)
        j = jnp.min(jnp.where(d == m, iota, n), axis=1, keepdims=True)
        out_ref[:, k:k + 1] = j
        d = jnp.where(iota == j, jnp.inf, d)


def _knn_select(sqr, nsample, *, s_tile=256):
    B, S, N = sqr.shape
    kfn = functools.partial(_knn_kernel, nsample=nsample, n=N)
    return pl.pallas_call(
        kfn,
        out_shape=jax.ShapeDtypeStruct((B, S, nsample), jnp.int32),
        grid=(B, S // s_tile),
        in_specs=[pl.BlockSpec((None, s_tile, N), lambda b, st: (b, st, 0))],
        out_specs=pl.BlockSpec((None, s_tile, nsample), lambda b, st: (b, st, 0)),
        compiler_params=pltpu.CompilerParams(
            dimension_semantics=("parallel", "parallel"),
            vmem_limit_bytes=_VMEM_LIMIT),
    )(sqr)


def _pointconv_kernel(x_ref, mw0_ref, mb0_ref, mw1_ref, mb1_ref,
                      vw0_ref, vb0_ref, vw1_ref, vb1_ref, vw2_ref, vb2_ref,
                      wl_ref, bl_ref, out_ref, *, k):
    skn, c_all = x_ref.shape
    s_t = skn // k
    cm = mw1_ref.shape[1]
    wc = vw2_ref.shape[1]

    x = x_ref[...]

    f = jnp.maximum(jnp.dot(x, mw0_ref[...],
                            preferred_element_type=jnp.float32) + mb0_ref[...], 0.0)
    f = jnp.maximum(jnp.dot(f, mw1_ref[...],
                            preferred_element_type=jnp.float32) + mb1_ref[...], 0.0)

    w = jnp.maximum(jnp.dot(x, vw0_ref[...],
                            preferred_element_type=jnp.float32) + vb0_ref[...], 0.0)
    w = jnp.maximum(jnp.dot(w, vw1_ref[...],
                            preferred_element_type=jnp.float32) + vb1_ref[...], 0.0)
    w = jnp.maximum(jnp.dot(w, vw2_ref[...],
                            preferred_element_type=jnp.float32) + vb2_ref[...], 0.0)

    wd = w * x[:, c_all - 1:c_all]

    f3t = jnp.swapaxes(f.reshape(s_t, k, cm), 1, 2)
    wd3 = wd.reshape(s_t, k, wc)
    agg = jnp.einsum("sck,skw->scw", f3t, wd3,
                     preferred_element_type=jnp.float32)

    flat = agg.reshape(s_t, cm * wc)
    out = jnp.dot(flat, wl_ref[...], preferred_element_type=jnp.float32) + bl_ref[...]
    out_ref[...] = jnp.maximum(out, 0.0)


def _pointconv(grouped_all, mlp, wnet, linear, *, s_tile=256):
    B, S, K, C_all = grouped_all.shape
    (mw0, mb0), (mw1, mb1) = mlp
    (vw0, vb0), (vw1, vb1), (vw2, vb2) = wnet
    wl, bl = linear
    CO = wl.shape[1]

    mw0_ext = jnp.concatenate(
        [mw0, jnp.zeros((C_all - mw0.shape[0], mw0.shape[1]), mw0.dtype)], axis=0)
    vw0_ext = jnp.concatenate(
        [vw0, jnp.zeros((C_all - vw0.shape[0], vw0.shape[1]), vw0.dtype)], axis=0)

    x_flat = grouped_all.reshape(B, S * K, C_all)
    kfn = functools.partial(_pointconv_kernel, k=K)

    def wspec(a):
        nd = a.ndim
        return pl.BlockSpec(a.shape, lambda b, st, nd=nd: (0,) * nd)

    return pl.pallas_call(
        kfn,
        out_shape=jax.ShapeDtypeStruct((B, S, CO), jnp.float32),
        grid=(B, S // s_tile),
        in_specs=[
            pl.BlockSpec((None, s_tile * K, C_all), lambda b, st: (b, st, 0)),
            wspec(mw0_ext), wspec(mb0), wspec(mw1), wspec(mb1),
            wspec(vw0_ext), wspec(vb0), wspec(vw1), wspec(vb1),
            wspec(vw2), wspec(vb2), wspec(wl), wspec(bl),
        ],
        out_specs=pl.BlockSpec((None, s_tile, CO), lambda b, st: (b, st, 0)),
        compiler_params=pltpu.CompilerParams(
            dimension_semantics=("parallel", "parallel"),
            vmem_limit_bytes=_VMEM_LIMIT),
    )(x_flat, mw0_ext, mb0, mw1, mb1, vw0_ext, vb0, vw1, vb1, vw2, vb2, wl, bl)


def _gather_points(points, idx):
    return jax.vmap(lambda p, i: p[i])(points, idx)


def kernel(xyz_in, pts_in,
           d0w, d0b, d1w, d1b, d2w, d2b,
           m0w, m0b, m1w, m1b,
           v0w, v0b, v1w, v1b, v2w, v2b,
           lw, lb):
    npoint, nsample, bandwidth = 512, 32, 0.5
    xyz = jnp.transpose(xyz_in, (0, 2, 1))
    points = jnp.transpose(pts_in, (0, 2, 1))

    density_scale = _density_scale(
        xyz, [(d0w, d0b), (d1w, d1b), (d2w, d2b)], bandwidth)

    fps_idx = _farthest_point_sample(xyz_in, npoint)
    new_xyz = _gather_points(xyz, fps_idx)

    sqr = -2.0 * jnp.einsum("bnc,bmc->bnm", new_xyz, xyz)
    sqr = sqr + jnp.sum(new_xyz ** 2, -1)[:, :, None]
    sqr = sqr + jnp.sum(xyz ** 2, -1)[:, None, :]
    idx = _knn_select(sqr, nsample)

    grouped_xyz = _gather_points(xyz, idx)
    grouped_xyz_norm = grouped_xyz - new_xyz[:, :, None, :]
    grouped_points = _gather_points(points, idx)
    grouped_density = _gather_points(density_scale, idx)
    grouped_all = jnp.concatenate(
        [grouped_xyz_norm, grouped_points, grouped_density], axis=-1)

    feat = _pointconv(grouped_all,
                      [(m0w, m0b), (m1w, m1b)],
                      [(v0w, v0b), (v1w, v1b), (v2w, v2b)],
                      (lw, lb))
    return jnp.transpose(new_xyz, (0, 2, 1)), jnp.transpose(feat, (0, 2, 1))


# speedup vs baseline: 4.3223x; 2.1061x over previous
import functools

import jax
import jax.numpy as jnp
from jax import lax
from jax.experimental import pallas as pl
from jax.experimental.pallas import tpu as pltpu

_VMEM_LIMIT = 48 * 1024 * 1024
_LOG2E = 1.4426950408889634


def _density_kernel(xyz_ref, tile_ref, w0_ref, b0_ref, w1_ref, b1_ref,
                    w2_ref, b2_ref, out_ref, *, nl2, post_scale):
    xa = xyz_ref[...]
    xr = tile_ref[...]
    xx = lax.dot_general(xr, xa, (((1,), (1,)), ((), ())),
                         preferred_element_type=jnp.float32)
    ra = jnp.sum(xr * xr, axis=-1, keepdims=True) * nl2
    aa = (jnp.sum(xa * xa, axis=-1) * nl2)[None, :]
    arg = jnp.minimum(ra + aa + (-2.0 * nl2) * xx, 0.0)
    g = jnp.exp2(arg)
    d = jnp.sum(g, axis=-1, keepdims=True) * post_scale

    h = jnp.maximum(d * w0_ref[...] + b0_ref[...], 0.0)
    h = jnp.maximum(jnp.dot(h, w1_ref[...],
                            preferred_element_type=jnp.float32) + b1_ref[...], 0.0)
    h = jnp.maximum(jnp.dot(h, w2_ref[...],
                            preferred_element_type=jnp.float32) + b2_ref[...], 0.0)
    out_ref[...] = h


def _density_scale(xyz, dnet, bandwidth, *, tn=512):
    B, N, _ = xyz.shape
    (w0, b0), (w1, b1), (w2, b2) = dnet
    neg = -1.0 / (2.0 * bandwidth * bandwidth)
    kfn = functools.partial(
        _density_kernel, nl2=neg * _LOG2E,
        post_scale=1.0 / (float(N) * 2.5 * bandwidth))

    def wspec(a):
        nd = a.ndim
        return pl.BlockSpec(a.shape, lambda b, t, nd=nd: (0,) * nd)

    return pl.pallas_call(
        kfn,
        out_shape=jax.ShapeDtypeStruct((B, N, 1), jnp.float32),
        grid=(B, N // tn),
        in_specs=[
            pl.BlockSpec((None, N, 3), lambda b, t: (b, 0, 0)),
            pl.BlockSpec((None, tn, 3), lambda b, t: (b, t, 0)),
            wspec(w0), wspec(b0), wspec(w1), wspec(b1), wspec(w2), wspec(b2),
        ],
        out_specs=pl.BlockSpec((None, tn, 1), lambda b, t: (b, t, 0)),
        compiler_params=pltpu.CompilerParams(
            dimension_semantics=("parallel", "parallel"),
            vmem_limit_bytes=_VMEM_LIMIT),
    )(xyz, xyz, w0, b0, w1, b1, w2, b2)


def _fps_kernel(xs_ref, ys_ref, zs_ref, out_ref, *, npoint, n):
    bt = xs_ref.shape[0]
    xs = xs_ref[...]
    ys = ys_ref[...]
    zs = zs_ref[...]
    iota = lax.broadcasted_iota(jnp.int32, (bt, n), 1)

    def body(i, carry):
        dmin, far = carry
        out_ref[pl.ds(i, 1), :, :] = far[None]
        mask = iota == far
        cx = jnp.sum(jnp.where(mask, xs, 0.0), axis=1, keepdims=True)
        cy = jnp.sum(jnp.where(mask, ys, 0.0), axis=1, keepdims=True)
        cz = jnp.sum(jnp.where(mask, zs, 0.0), axis=1, keepdims=True)
        dx = xs - cx
        dy = ys - cy
        dz = zs - cz
        dist = (dx * dx + dz * dz) + dy * dy
        dmin = jnp.minimum(dmin, dist)
        m = jnp.max(dmin, axis=1, keepdims=True)
        far = jnp.min(jnp.where(dmin == m, iota, n), axis=1, keepdims=True)
        return dmin, far

    dmin0 = jnp.full((bt, n), 1e10, jnp.float32)
    far0 = jnp.zeros((bt, 1), jnp.int32)
    lax.fori_loop(0, npoint, body, (dmin0, far0))


def _farthest_point_sample(xyz_b3n, npoint, *, bt=8):
    B, _, N = xyz_b3n.shape
    kfn = functools.partial(_fps_kernel, npoint=npoint, n=N)
    out = pl.pallas_call(
        kfn,
        out_shape=jax.ShapeDtypeStruct((B // bt, npoint, bt, 1), jnp.int32),
        grid=(B // bt,),
        in_specs=[pl.BlockSpec((bt, N), lambda g: (g, 0))] * 3,
        out_specs=pl.BlockSpec((None, npoint, bt, 1), lambda g: (g, 0, 0, 0)),
        compiler_params=pltpu.CompilerParams(
            dimension_semantics=("parallel",),
            vmem_limit_bytes=_VMEM_LIMIT),
    )(xyz_b3n[:, 0, :], xyz_b3n[:, 1, :], xyz_b3n[:, 2, :])
    return out[..., 0].transpose(0, 2, 1).reshape(B, npoint)


def _knn_kernel(sqr_ref, out_ref, *, nsample, n):
    d = sqr_ref[...]
    st = d.shape[0]
    iota = lax.broadcasted_iota(jnp.int32, (st, n), 1)
    for k in range(nsample):
        m = jnp.min(d, axis=1, keepdims=True)
        j = jnp.min(jnp.where(d == m, iota, n), axis=1, keepdims=True)
        out_ref[:, k:k + 1] = j
        d = jnp.where(iota == j, jnp.inf, d)


def _knn_select(sqr, nsample, *, s_tile=256):
    B, S, N = sqr.shape
    kfn = functools.partial(_knn_kernel, nsample=nsample, n=N)
    return pl.pallas_call(
        kfn,
        out_shape=jax.ShapeDtypeStruct((B, S, nsample), jnp.int32),
        grid=(B, S // s_tile),
        in_specs=[pl.BlockSpec((None, s_tile, N), lambda b, st: (b, st, 0))],
        out_specs=pl.BlockSpec((None, s_tile, nsample), lambda b, st: (b, st, 0)),
        compiler_params=pltpu.CompilerParams(
            dimension_semantics=("parallel", "parallel"),
            vmem_limit_bytes=_VMEM_LIMIT),
    )(sqr)


def _pointconv_kernel(x_ref, bm_ref, bw_ref, mw0_ref, mb0_ref, mw1_ref, mb1_ref,
                      vw0_ref, vb0_ref, vw1_ref, vb1_ref, vw2_ref, vb2_ref,
                      wl_ref, bl_ref, out_ref, *, k):
    skn, c_all = x_ref.shape
    s_t = skn // k
    cm = mw1_ref.shape[1]
    wc = vw2_ref.shape[1]

    x = x_ref[...]

    f = jnp.dot(x, mw0_ref[...], preferred_element_type=jnp.float32) + mb0_ref[...]
    f = jnp.maximum(
        (f.reshape(s_t, k, cm) - bm_ref[...][:, None, :]).reshape(skn, cm), 0.0)
    f = jnp.maximum(jnp.dot(f, mw1_ref[...],
                            preferred_element_type=jnp.float32) + mb1_ref[...], 0.0)

    w = jnp.dot(x, vw0_ref[...], preferred_element_type=jnp.float32) + vb0_ref[...]
    h0 = vw1_ref.shape[0]
    w = jnp.maximum(
        (w.reshape(s_t, k, h0) - bw_ref[...][:, None, :]).reshape(skn, h0), 0.0)
    w = jnp.maximum(jnp.dot(w, vw1_ref[...],
                            preferred_element_type=jnp.float32) + vb1_ref[...], 0.0)
    w = jnp.maximum(jnp.dot(w, vw2_ref[...],
                            preferred_element_type=jnp.float32) + vb2_ref[...], 0.0)

    wd = w * x[:, c_all - 1:c_all]

    f3t = jnp.swapaxes(f.reshape(s_t, k, cm), 1, 2)
    wd3 = wd.reshape(s_t, k, wc)
    agg = jnp.einsum("sck,skw->scw", f3t, wd3,
                     preferred_element_type=jnp.float32)

    flat = agg.reshape(s_t, cm * wc)
    out = jnp.dot(flat, wl_ref[...], preferred_element_type=jnp.float32) + bl_ref[...]
    out_ref[...] = jnp.maximum(out, 0.0)


def _pointconv(grouped_all, new_xyz, mlp, wnet, linear, *, s_tile=256):
    B, S, K, C_all = grouped_all.shape
    (mw0, mb0), (mw1, mb1) = mlp
    (vw0, vb0), (vw1, vb1), (vw2, vb2) = wnet
    wl, bl = linear
    CO = wl.shape[1]

    mw0_ext = jnp.concatenate(
        [mw0, jnp.zeros((C_all - mw0.shape[0], mw0.shape[1]), mw0.dtype)], axis=0)
    vw0_ext = jnp.concatenate(
        [vw0, jnp.zeros((C_all - vw0.shape[0], vw0.shape[1]), vw0.dtype)], axis=0)

    bm = jnp.einsum("bsc,cm->bsm", new_xyz, mw0[:3])
    bw = jnp.einsum("bsc,cm->bsm", new_xyz, vw0[:3])

    x_flat = grouped_all.reshape(B, S * K, C_all)
    kfn = functools.partial(_pointconv_kernel, k=K)

    def wspec(a):
        nd = a.ndim
        return pl.BlockSpec(a.shape, lambda b, st, nd=nd: (0,) * nd)

    return pl.pallas_call(
        kfn,
        out_shape=jax.ShapeDtypeStruct((B, S, CO), jnp.float32),
        grid=(B, S // s_tile),
        in_specs=[
            pl.BlockSpec((None, s_tile * K, C_all), lambda b, st: (b, st, 0)),
            pl.BlockSpec((None, s_tile, bm.shape[2]), lambda b, st: (b, st, 0)),
            pl.BlockSpec((None, s_tile, bw.shape[2]), lambda b, st: (b, st, 0)),
            wspec(mw0_ext), wspec(mb0), wspec(mw1), wspec(mb1),
            wspec(vw0_ext), wspec(vb0), wspec(vw1), wspec(vb1),
            wspec(vw2), wspec(vb2), wspec(wl), wspec(bl),
        ],
        out_specs=pl.BlockSpec((None, s_tile, CO), lambda b, st: (b, st, 0)),
        compiler_params=pltpu.CompilerParams(
            dimension_semantics=("parallel", "parallel"),
            vmem_limit_bytes=_VMEM_LIMIT),
    )(x_flat, bm, bw, mw0_ext, mb0, mw1, mb1, vw0_ext, vb0,
      vw1, vb1, vw2, vb2, wl, bl)


def _gather_points(points, idx):
    return jax.vmap(lambda p, i: p[i])(points, idx)


def kernel(xyz_in, pts_in,
           d0w, d0b, d1w, d1b, d2w, d2b,
           m0w, m0b, m1w, m1b,
           v0w, v0b, v1w, v1b, v2w, v2b,
           lw, lb):
    npoint, nsample, bandwidth = 512, 32, 0.5
    xyz = jnp.transpose(xyz_in, (0, 2, 1))
    points = jnp.transpose(pts_in, (0, 2, 1))

    density_scale = _density_scale(
        xyz, [(d0w, d0b), (d1w, d1b), (d2w, d2b)], bandwidth)

    fps_idx = _farthest_point_sample(xyz_in, npoint)
    new_xyz = _gather_points(xyz, fps_idx)

    sqr = -2.0 * jnp.einsum("bnc,bmc->bnm", new_xyz, xyz)
    sqr = sqr + jnp.sum(new_xyz ** 2, -1)[:, :, None]
    sqr = sqr + jnp.sum(xyz ** 2, -1)[:, None, :]
    idx = _knn_select(sqr, nsample)

    allp = jnp.concatenate([xyz, points, density_scale], axis=-1)
    grouped_all = _gather_points(allp, idx)

    feat = _pointconv(grouped_all, new_xyz,
                      [(m0w, m0b), (m1w, m1b)],
                      [(v0w, v0b), (v1w, v1b), (v2w, v2b)],
                      (lw, lb))
    return jnp.transpose(new_xyz, (0, 2, 1)), jnp.transpose(feat, (0, 2, 1))


# speedup vs baseline: 4.4084x; 1.0199x over previous
import functools

import jax
import jax.numpy as jnp
from jax import lax
from jax.experimental import pallas as pl
from jax.experimental.pallas import tpu as pltpu

_VMEM_LIMIT = 48 * 1024 * 1024
_LOG2E = 1.4426950408889634


def _density_kernel(xyz_ref, tile_ref, w0_ref, b0_ref, w1_ref, b1_ref,
                    w2_ref, b2_ref, out_ref, *, nl2, post_scale):
    xa = xyz_ref[...]
    xr = tile_ref[...]
    xx = lax.dot_general(xr, xa, (((1,), (1,)), ((), ())),
                         preferred_element_type=jnp.float32)
    ra = jnp.sum(xr * xr, axis=-1, keepdims=True) * nl2
    aa = (jnp.sum(xa * xa, axis=-1) * nl2)[None, :]
    arg = jnp.minimum(ra + aa + (-2.0 * nl2) * xx, 0.0)
    g = jnp.exp2(arg)
    d = jnp.sum(g, axis=-1, keepdims=True) * post_scale

    h = jnp.maximum(d * w0_ref[...] + b0_ref[...], 0.0)
    h = jnp.maximum(jnp.dot(h, w1_ref[...],
                            preferred_element_type=jnp.float32) + b1_ref[...], 0.0)
    h = jnp.maximum(jnp.dot(h, w2_ref[...],
                            preferred_element_type=jnp.float32) + b2_ref[...], 0.0)
    out_ref[...] = h


def _density_scale(xyz, dnet, bandwidth, *, tn=512):
    B, N, _ = xyz.shape
    (w0, b0), (w1, b1), (w2, b2) = dnet
    neg = -1.0 / (2.0 * bandwidth * bandwidth)
    kfn = functools.partial(
        _density_kernel, nl2=neg * _LOG2E,
        post_scale=1.0 / (float(N) * 2.5 * bandwidth))

    def wspec(a):
        nd = a.ndim
        return pl.BlockSpec(a.shape, lambda b, t, nd=nd: (0,) * nd)

    return pl.pallas_call(
        kfn,
        out_shape=jax.ShapeDtypeStruct((B, N, 1), jnp.float32),
        grid=(B, N // tn),
        in_specs=[
            pl.BlockSpec((None, N, 3), lambda b, t: (b, 0, 0)),
            pl.BlockSpec((None, tn, 3), lambda b, t: (b, t, 0)),
            wspec(w0), wspec(b0), wspec(w1), wspec(b1), wspec(w2), wspec(b2),
        ],
        out_specs=pl.BlockSpec((None, tn, 1), lambda b, t: (b, t, 0)),
        compiler_params=pltpu.CompilerParams(
            dimension_semantics=("parallel", "parallel"),
            vmem_limit_bytes=_VMEM_LIMIT),
    )(xyz, xyz, w0, b0, w1, b1, w2, b2)


def _fps_kernel(xs_ref, ys_ref, zs_ref, out_ref, *, npoint, n):
    bt = xs_ref.shape[0]
    xs = xs_ref[...]
    ys = ys_ref[...]
    zs = zs_ref[...]
    iota = lax.broadcasted_iota(jnp.int32, (bt, n), 1)

    def body(i, carry):
        dmin, far = carry
        out_ref[pl.ds(i, 1), :, :] = far[None]
        mask = iota == far
        cx = jnp.sum(jnp.where(mask, xs, 0.0), axis=1, keepdims=True)
        cy = jnp.sum(jnp.where(mask, ys, 0.0), axis=1, keepdims=True)
        cz = jnp.sum(jnp.where(mask, zs, 0.0), axis=1, keepdims=True)
        dx = xs - cx
        dy = ys - cy
        dz = zs - cz
        dist = (dx * dx + dz * dz) + dy * dy
        dmin = jnp.minimum(dmin, dist)
        m = jnp.max(dmin, axis=1, keepdims=True)
        far = jnp.min(jnp.where(dmin == m, iota, n), axis=1, keepdims=True)
        return dmin, far

    dmin0 = jnp.full((bt, n), 1e10, jnp.float32)
    far0 = jnp.zeros((bt, 1), jnp.int32)
    lax.fori_loop(0, npoint, body, (dmin0, far0))


def _farthest_point_sample(xyz_b3n, npoint, *, bt=8):
    B, _, N = xyz_b3n.shape
    kfn = functools.partial(_fps_kernel, npoint=npoint, n=N)
    out = pl.pallas_call(
        kfn,
        out_shape=jax.ShapeDtypeStruct((B // bt, npoint, bt, 1), jnp.int32),
        grid=(B // bt,),
        in_specs=[pl.BlockSpec((bt, N), lambda g: (g, 0))] * 3,
        out_specs=pl.BlockSpec((None, npoint, bt, 1), lambda g: (g, 0, 0, 0)),
        compiler_params=pltpu.CompilerParams(
            dimension_semantics=("parallel",),
            vmem_limit_bytes=_VMEM_LIMIT),
    )(xyz_b3n[:, 0, :], xyz_b3n[:, 1, :], xyz_b3n[:, 2, :])
    return out[..., 0].transpose(0, 2, 1).reshape(B, npoint)


def _knn_kernel(sqr_ref, out_ref, *, nsample, n):
    d = sqr_ref[...]
    st = d.shape[0]
    iota = lax.broadcasted_iota(jnp.int32, (st, n), 1)
    for k in range(nsample):
        m = jnp.min(d, axis=1, keepdims=True)
        j = jnp.min(jnp.where(d == m, iota, n), axis=1, keepdims=True)
        out_ref[:, k:k + 1] = j
        d = jnp.where(iota == j, jnp.inf, d)


def _knn_select(sqr, nsample, *, s_tile=256):
    B, S, N = sqr.shape
    kfn = functools.partial(_knn_kernel, nsample=nsample, n=N)
    return pl.pallas_call(
        kfn,
        out_shape=jax.ShapeDtypeStruct((B, S, nsample), jnp.int32),
        grid=(B, S // s_tile),
        in_specs=[pl.BlockSpec((None, s_tile, N), lambda b, st: (b, st, 0))],
        out_specs=pl.BlockSpec((None, s_tile, nsample), lambda b, st: (b, st, 0)),
        compiler_params=pltpu.CompilerParams(
            dimension_semantics=("parallel", "parallel"),
            vmem_limit_bytes=_VMEM_LIMIT),
    )(sqr)


def _pointconv_kernel(x_ref, bm_ref, bw_ref, mw0_ref, mb0_ref, mw1_ref, mb1_ref,
                      vw0_ref, vb0_ref, vw1_ref, vb1_ref, vw2_ref, vb2_ref,
                      wl_ref, bl_ref, out_ref, *, k):
    skn, c_all = x_ref.shape
    s_t = skn // k
    cm = mw1_ref.shape[1]
    wc = vw2_ref.shape[1]

    x = x_ref[...]

    f = jnp.dot(x, mw0_ref[...], preferred_element_type=jnp.float32) + mb0_ref[...]
    f = jnp.maximum(
        (f.reshape(s_t, k, cm) - bm_ref[...][:, None, :]).reshape(skn, cm), 0.0)
    f = jnp.maximum(jnp.dot(f, mw1_ref[...],
                            preferred_element_type=jnp.float32) + mb1_ref[...], 0.0)

    w = jnp.dot(x, vw0_ref[...], preferred_element_type=jnp.float32) + vb0_ref[...]
    h0 = vw1_ref.shape[0]
    w = jnp.maximum(
        (w.reshape(s_t, k, h0) - bw_ref[...][:, None, :]).reshape(skn, h0), 0.0)
    w = jnp.maximum(jnp.dot(w, vw1_ref[...],
                            preferred_element_type=jnp.float32) + vb1_ref[...], 0.0)
    w = jnp.maximum(jnp.dot(w, vw2_ref[...],
                            preferred_element_type=jnp.float32) + vb2_ref[...], 0.0)

    wd = w * x[:, c_all - 1:c_all]

    f3t = jnp.swapaxes(f.reshape(s_t, k, cm), 1, 2)
    wd3 = wd.reshape(s_t, k, wc)
    agg = jnp.einsum("sck,skw->scw", f3t, wd3,
                     preferred_element_type=jnp.float32)

    flat = agg.reshape(s_t, cm * wc)
    out = jnp.dot(flat, wl_ref[...], preferred_element_type=jnp.float32) + bl_ref[...]
    out_ref[...] = jnp.maximum(out, 0.0)


def _pointconv(grouped_all, new_xyz, mlp, wnet, linear, *, s_tile=256):
    B, S, K, C_all = grouped_all.shape
    (mw0, mb0), (mw1, mb1) = mlp
    (vw0, vb0), (vw1, vb1), (vw2, vb2) = wnet
    wl, bl = linear
    CO = wl.shape[1]

    mw0_ext = jnp.concatenate(
        [mw0, jnp.zeros((C_all - mw0.shape[0], mw0.shape[1]), mw0.dtype)], axis=0)
    vw0_ext = jnp.concatenate(
        [vw0, jnp.zeros((C_all - vw0.shape[0], vw0.shape[1]), vw0.dtype)], axis=0)

    bm = jnp.einsum("bsc,cm->bsm", new_xyz, mw0[:3])
    bw = jnp.einsum("bsc,cm->bsm", new_xyz, vw0[:3])

    x_flat = grouped_all.reshape(B, S * K, C_all)
    kfn = functools.partial(_pointconv_kernel, k=K)

    def wspec(a):
        nd = a.ndim
        return pl.BlockSpec(a.shape, lambda b, st, nd=nd: (0,) * nd)

    return pl.pallas_call(
        kfn,
        out_shape=jax.ShapeDtypeStruct((B, S, CO), jnp.float32),
        grid=(B, S // s_tile),
        in_specs=[
            pl.BlockSpec((None, s_tile * K, C_all), lambda b, st: (b, st, 0)),
            pl.BlockSpec((None, s_tile, bm.shape[2]), lambda b, st: (b, st, 0)),
            pl.BlockSpec((None, s_tile, bw.shape[2]), lambda b, st: (b, st, 0)),
            wspec(mw0_ext), wspec(mb0), wspec(mw1), wspec(mb1),
            wspec(vw0_ext), wspec(vb0), wspec(vw1), wspec(vb1),
            wspec(vw2), wspec(vb2), wspec(wl), wspec(bl),
        ],
        out_specs=pl.BlockSpec((None, s_tile, CO), lambda b, st: (b, st, 0)),
        compiler_params=pltpu.CompilerParams(
            dimension_semantics=("parallel", "parallel"),
            vmem_limit_bytes=_VMEM_LIMIT),
    )(x_flat, bm, bw, mw0_ext, mb0, mw1, mb1, vw0_ext, vb0,
      vw1, vb1, vw2, vb2, wl, bl)


def _gather_points(points, idx):
    return jax.vmap(lambda p, i: p[i])(points, idx)


def kernel(xyz_in, pts_in,
           d0w, d0b, d1w, d1b, d2w, d2b,
           m0w, m0b, m1w, m1b,
           v0w, v0b, v1w, v1b, v2w, v2b,
           lw, lb):
    npoint, nsample, bandwidth = 512, 32, 0.5
    xyz = jnp.transpose(xyz_in, (0, 2, 1))
    points = jnp.transpose(pts_in, (0, 2, 1))

    density_scale = _density_scale(
        xyz, [(d0w, d0b), (d1w, d1b), (d2w, d2b)], bandwidth)

    fps_idx = _farthest_point_sample(xyz_in, npoint)
    new_xyz = _gather_points(xyz, fps_idx)

    nxb, xyzb = lax.optimization_barrier((new_xyz, xyz))
    sqr = -2.0 * jnp.einsum("bnc,bmc->bnm", nxb, xyzb)
    sqr = sqr + jnp.sum(nxb ** 2, -1)[:, :, None]
    sqr = sqr + jnp.sum(xyzb ** 2, -1)[:, None, :]
    idx = _knn_select(sqr, nsample)

    allp = jnp.concatenate([xyz, points, density_scale], axis=-1)
    grouped_all = _gather_points(allp, idx)

    feat = _pointconv(grouped_all, new_xyz,
                      [(m0w, m0b), (m1w, m1b)],
                      [(v0w, v0b), (v1w, v1b), (v2w, v2b)],
                      (lw, lb))
    return jnp.transpose(new_xyz, (0, 2, 1)), jnp.transpose(feat, (0, 2, 1))
